# baseline (device time: 40112 ns/iter reference)
import jax
import jax.numpy as jnp
from jax import lax
from jax.experimental import pallas as pl
from jax.experimental.pallas import tpu as pltpu

N_DEV = 32
QCLIP = 4.5
QSCALE = 127.0 / QCLIP


def kernel(q, k, v):
    m_per, d = q.shape
    s_total = N_DEV * m_per
    scale = 1.0 / float(d) ** 0.5

    def body(q_ref, k_ref, v_ref, out_ref, hbm_ref, kv_vm,
             send_sems, recv_sems, ready_sems, copy_sems):
        my = lax.axis_index("i")

        barrier_sem = pltpu.get_barrier_semaphore()
        pl.semaphore_signal(
            barrier_sem, inc=1,
            device_id=(lax.rem(my + 1, N_DEV),),
            device_id_type=pl.DeviceIdType.MESH,
        )
        for dd in range(1, N_DEV):
            peer = lax.rem(my + dd, N_DEV)
            pl.semaphore_signal(
                ready_sems.at[my], inc=1,
                device_id=(peer,), device_id_type=pl.DeviceIdType.MESH,
            )
        pl.semaphore_wait(barrier_sem, 1)

        def quant(x):
            return jnp.clip(
                jnp.round(x * QSCALE), -127.0, 127.0
            ).astype(jnp.int8)

        kv_vm[pl.ds(my, 1), 0, :, :] = quant(k_ref[:, :])[None]
        kv_vm[pl.ds(my, 1), 1, :, :] = quant(v_ref[:, :])[None]

        for dd in range(1, N_DEV):
            peer = lax.rem(my + dd, N_DEV)
            dst = kv_vm if dd % 2 == 1 else hbm_ref
            pl.semaphore_wait(ready_sems.at[peer], 1)
            pltpu.make_async_remote_copy(
                src_ref=kv_vm.at[my], dst_ref=dst.at[my],
                send_sem=send_sems.at[dd - 1], recv_sem=recv_sems.at[my],
                device_id=(peer,), device_id_type=pl.DeviceIdType.MESH,
            ).start()

        copies = []
        for dd in range(1, N_DEV):
            src = lax.rem(my - dd + N_DEV, N_DEV)
            landed = kv_vm if dd % 2 == 1 else hbm_ref
            pltpu.make_async_remote_copy(
                src_ref=kv_vm.at[my], dst_ref=landed.at[src],
                send_sem=send_sems.at[dd - 1], recv_sem=recv_sems.at[src],
                device_id=(src,), device_id_type=pl.DeviceIdType.MESH,
            ).wait_recv()
            if dd % 2 == 0:
                cp = pltpu.make_async_copy(
                    hbm_ref.at[src], kv_vm.at[src],
                    copy_sems.at[dd // 2 - 1],
                )
                cp.start()
                copies.append(cp)
        for cp in copies:
            cp.wait()

        kv = kv_vm[:, :, :, :]
        k_full = kv[:, 0].reshape(s_total, d).astype(jnp.bfloat16)
        v_full = kv[:, 1].reshape(s_total, d).astype(jnp.bfloat16)
        qb = q_ref[:, :].astype(jnp.bfloat16)
        s = lax.dot_general(
            qb, k_full, (((1,), (1,)), ((), ())),
            preferred_element_type=jnp.float32,
        ) * (scale / QSCALE)
        m = jnp.max(s, axis=1, keepdims=True)
        p = jnp.exp(s - m)
        l = jnp.sum(p, axis=1, keepdims=True)
        o = lax.dot_general(
            p.astype(jnp.bfloat16), v_full, (((1,), (0,)), ((), ())),
            preferred_element_type=jnp.float32,
        )
        out_ref[:, :] = o / (l * QSCALE)

        for dd in range(1, N_DEV):
            peer = lax.rem(my + dd, N_DEV)
            dst = kv_vm if dd % 2 == 1 else hbm_ref
            pltpu.make_async_remote_copy(
                src_ref=kv_vm.at[my], dst_ref=dst.at[my],
                send_sem=send_sems.at[dd - 1], recv_sem=recv_sems.at[my],
                device_id=(peer,), device_id_type=pl.DeviceIdType.MESH,
            ).wait_send()

    return pl.pallas_call(
        body,
        out_shape=(
            jax.ShapeDtypeStruct((m_per, d), jnp.float32),
            jax.ShapeDtypeStruct((N_DEV, 2, m_per, d), jnp.int8),
        ),
        in_specs=[pl.BlockSpec(memory_space=pltpu.VMEM)] * 3,
        out_specs=(
            pl.BlockSpec(memory_space=pltpu.VMEM),
            pl.BlockSpec(memory_space=pltpu.MemorySpace.HBM),
        ),
        scratch_shapes=[
            pltpu.VMEM((N_DEV, 2, m_per, d), jnp.int8),
            pltpu.SemaphoreType.DMA((N_DEV - 1,)),
            pltpu.SemaphoreType.DMA((N_DEV,)),
            pltpu.SemaphoreType.REGULAR((N_DEV,)),
            pltpu.SemaphoreType.DMA((N_DEV // 2 - 1,)),
        ],
        compiler_params=pltpu.CompilerParams(collective_id=0),
    )(q, k, v)[0]


# device time: 37915 ns/iter; 1.0579x vs baseline; 1.0579x over previous
import jax
import jax.numpy as jnp
from jax import lax
from jax.experimental import pallas as pl
from jax.experimental.pallas import tpu as pltpu

N_DEV = 32
QCLIP = 4.5
QSCALE = 127.0 / QCLIP


def kernel(q, k, v):
    m_per, d = q.shape
    s_total = N_DEV * m_per
    scale = 1.0 / float(d) ** 0.5

    def body(q_ref, k_ref, v_ref, out_ref, kv_all,
             send_sems, recv_sems, ready_sems):
        my = lax.axis_index("i")

        barrier_sem = pltpu.get_barrier_semaphore()
        pl.semaphore_signal(
            barrier_sem, inc=1,
            device_id=(lax.rem(my + 1, N_DEV),),
            device_id_type=pl.DeviceIdType.MESH,
        )
        for dd in range(1, N_DEV):
            peer = lax.rem(my + dd, N_DEV)
            pl.semaphore_signal(
                ready_sems.at[my], inc=1,
                device_id=(peer,), device_id_type=pl.DeviceIdType.MESH,
            )
        pl.semaphore_wait(barrier_sem, 1)

        def quant(x):
            return jnp.clip(
                jnp.round(x * QSCALE), -127.0, 127.0
            ).astype(jnp.int8)

        kv_all[pl.ds(my, 1), 0, :, :] = quant(k_ref[:, :])[None]
        kv_all[pl.ds(my, 1), 1, :, :] = quant(v_ref[:, :])[None]

        for dd in range(1, N_DEV):
            peer = lax.rem(my + dd, N_DEV)
            pl.semaphore_wait(ready_sems.at[peer], 1)
            pltpu.make_async_remote_copy(
                src_ref=kv_all.at[my], dst_ref=kv_all.at[my],
                send_sem=send_sems.at[dd - 1], recv_sem=recv_sems.at[my],
                device_id=(peer,), device_id_type=pl.DeviceIdType.MESH,
            ).start()

        for dd in range(1, N_DEV):
            src = lax.rem(my + dd, N_DEV)
            pltpu.make_async_remote_copy(
                src_ref=kv_all.at[src], dst_ref=kv_all.at[src],
                send_sem=send_sems.at[dd - 1], recv_sem=recv_sems.at[src],
                device_id=(src,), device_id_type=pl.DeviceIdType.MESH,
            ).wait_recv()

        kv = kv_all[:, :, :, :]
        k_full = kv[:, 0].reshape(s_total, d).astype(jnp.bfloat16)
        v_full = kv[:, 1].reshape(s_total, d).astype(jnp.bfloat16)
        qb = q_ref[:, :].astype(jnp.bfloat16)
        s = lax.dot_general(
            qb, k_full, (((1,), (1,)), ((), ())),
            preferred_element_type=jnp.float32,
        ) * (scale / QSCALE)
        m = jnp.max(s, axis=1, keepdims=True)
        p = jnp.exp(s - m)
        l = jnp.sum(p, axis=1, keepdims=True)
        o = lax.dot_general(
            p.astype(jnp.bfloat16), v_full, (((1,), (0,)), ((), ())),
            preferred_element_type=jnp.float32,
        )
        out_ref[:, :] = o / (l * QSCALE)

        for dd in range(1, N_DEV):
            peer = lax.rem(my + dd, N_DEV)
            pltpu.make_async_remote_copy(
                src_ref=kv_all.at[my], dst_ref=kv_all.at[my],
                send_sem=send_sems.at[dd - 1], recv_sem=recv_sems.at[my],
                device_id=(peer,), device_id_type=pl.DeviceIdType.MESH,
            ).wait_send()

    return pl.pallas_call(
        body,
        out_shape=jax.ShapeDtypeStruct((m_per, d), jnp.float32),
        in_specs=[pl.BlockSpec(memory_space=pltpu.VMEM)] * 3,
        out_specs=pl.BlockSpec(memory_space=pltpu.VMEM),
        scratch_shapes=[
            pltpu.VMEM((N_DEV, 2, m_per, d), jnp.int8),
            pltpu.SemaphoreType.DMA((N_DEV - 1,)),
            pltpu.SemaphoreType.DMA((N_DEV,)),
            pltpu.SemaphoreType.REGULAR((N_DEV,)),
        ],
        compiler_params=pltpu.CompilerParams(collective_id=0),
    )(q, k, v)
